# baseline (speedup 1.0000x reference)
import functools

import jax
import jax.numpy as jnp
from jax import lax
from jax.experimental import pallas as pl
from jax.experimental.pallas import tpu as pltpu
from jax.experimental.pallas import tpu_sc as plsc

X_MIN, X_MAX = -51.2, 51.2
Y_MIN, Y_MAX = -51.2, 51.2
Z_MIN = -2.0
STEP = 0.2
D, H, W = 30, 512, 512
B, N = 4, 131072
G = D * H * W
TOTAL = B * G
NTILE = (B * N) // 32
CH = 4096
NCH = NTILE // CH
ZCH = 8192
ZPER = G // 8
NZ = ZPER // ZCH
ROWS = NTILE // 128

RH = 48
RW = 48
Y0 = 232
X0 = 232
RROWS = D * RH
RGW = RROWS * RW
TRASH = RGW
WOUT = (D * RH) // 8
GTRASH = ROWS + 8

_mesh = plsc.VectorSubcoreMesh(core_axis_name="c", subcore_axis_name="s")


@functools.partial(
    pl.kernel,
    out_type=(jax.ShapeDtypeStruct((TOTAL,), jnp.float32),
              jax.ShapeDtypeStruct((256 * WOUT * RW,), jnp.float32)),
    mesh=_mesh,
    scratch_types=[
        pltpu.VMEM((2 * CH * 3,), jnp.float32),
        pltpu.VMEM((RGW + 16,), jnp.float32),
        pltpu.VMEM((ROWS, 128), jnp.int32),
        pltpu.VMEM((ZCH,), jnp.float32),
        pltpu.VMEM((ROWS + 16,), jnp.int32),
        pltpu.VMEM((WOUT * RW,), jnp.float32),
        pltpu.VMEM((128,), jnp.float32),
        pltpu.VMEM((16,), jnp.int32),
        pltpu.VMEM((16,), jnp.int32),
        pltpu.SemaphoreType.DMA,
        pltpu.SemaphoreType.DMA,
        pltpu.SemaphoreType.DMA,
    ],
    compiler_params=pltpu.CompilerParams(needs_layout_passes=False),
)
def _voxelize(pts, zsrc, out, mscr, pb, bbgrid, idxb, zbuf, gflag, tmpb,
              ones, flagbuf, fbbuf, sem_p, sem_z, sem_s):
    c = lax.axis_index("c")
    s = lax.axis_index("s")
    b = 2 * c + s // 8
    k = s % 8
    slot = s // 8
    pbase = k * NTILE
    base_cell = b * G

    cp_0 = pltpu.async_copy(zsrc, zbuf, sem_z)
    pdesc = [None] * NCH
    pdesc[0] = pltpu.async_copy(
        pts.at[pl.ds((b * N + pbase) * 3, CH * 3)],
        pb.at[pl.ds(0, CH * 3)], sem_p)

    lanes = lax.iota(jnp.int32, 16)
    zeroi = jnp.full((16,), 0, jnp.int32)
    onei = jnp.full((16,), 1, jnp.int32)
    onev = jnp.ones((16,), jnp.float32)
    zerov = jnp.zeros((16,), jnp.float32)
    for i in range(8):
        ones[pl.ds(i * 16, 16)] = onev
    flagbuf[...] = zeroi
    fbbuf[...] = zeroi
    for i in range((ROWS + 16) // 16):
        gflag[pl.ds(i * 16, 16)] = zeroi

    @plsc.parallel_loop(0, (RGW + 16) // 16, unroll=8)
    def bbzero(i):
        bbgrid[pl.ds(i * 16, 16)] = zerov

    cp_0.wait()
    zstart = base_cell + k * ZPER
    zdescs = [
        pltpu.async_copy(zbuf, out.at[pl.ds(zstart + i * ZCH, ZCH)], sem_z)
        for i in range(NZ)
    ]

    lanes3 = lanes * 3
    sent = jnp.full((16,), -1, jnp.int32)
    trash = jnp.full((16,), TRASH, jnp.int32)
    gtrash = jnp.full((16,), GTRASH, jnp.int32)

    for cc in range(NCH):
        if cc + 1 < NCH:
            pdesc[cc + 1] = pltpu.async_copy(
                pts.at[pl.ds((b * N + pbase + (cc + 1) * CH) * 3, CH * 3)],
                pb.at[pl.ds(((cc + 1) % 2) * (CH * 3), CH * 3)], sem_p)
        pdesc[cc].wait()
        pbv = pb
        boff = (cc % 2) * (CH * 3)

        @plsc.parallel_loop(0, CH // 16, unroll=4)
        def sweep1(jj):
            xi = lanes3 + (jj * 48 + boff)
            yi = xi + onei
            zi = yi + onei
            x = plsc.load_gather(pbv, [xi])
            y = plsc.load_gather(pbv, [yi])
            z = plsc.load_gather(pbv, [zi])
            bx = jnp.clip((x - X_MIN) / STEP, 0.0,
                          float(W - 1)).astype(jnp.int32)
            by = jnp.clip((Y_MAX - y) / STEP, 0.0,
                          float(H - 1)).astype(jnp.int32)
            bz = jnp.clip((z - Z_MIN) / STEP, 0.0,
                          float(D - 1)).astype(jnp.int32)
            m = (x >= X_MIN) & (x <= X_MAX) & (y >= Y_MIN) & (y <= Y_MAX)
            idx = base_cell + ((bz << 18) + (by << 9) + bx)
            idx = jnp.clip(idx, 0, TOTAL - 1)
            ry = by - Y0
            rx = bx - X0
            inside = (m & (ry >= zeroi) & (ry < jnp.full((16,), RH, jnp.int32))
                      & (rx >= zeroi) & (rx < jnp.full((16,), RW, jnp.int32)))
            rg = jnp.clip((bz * RH + ry) * RW + rx, 0, TRASH)
            plsc.store_scatter(bbgrid, [jnp.where(inside, rg, trash)], onev)
            outlier = m & jnp.logical_not(inside)
            jg = cc * (CH // 16) + jj
            g = jg // 8
            gv = jnp.full((16,), g, jnp.int32)
            plsc.store_scatter(gflag, [jnp.where(outlier, gv, gtrash)], onei)
            idxb[g, pl.ds((jg % 8) * 16, 16)] = jnp.where(m, idx, sent)
            av = jnp.where(m, zeroi, onei)
            plsc.store_scatter(flagbuf, [av], onei)
            plsc.store_scatter(fbbuf, [av], idx)

    fb = plsc.load_gather(fbbuf, [zeroi])

    @plsc.parallel_loop(0, NTILE // 16, unroll=8)
    def sweep2(j):
        r = j // 8
        colw = (j % 8) * 16
        v = idxb[r, pl.ds(colw, 16)]
        idxb[r, pl.ds(colw, 16)] = jnp.where(v < zeroi, fb, v)

    tid = c * 16 + s
    for kk in range(8):
        pltpu.sync_copy(bbgrid.at[pl.ds(kk * WOUT * RW, WOUT * RW)],
                        mscr.at[pl.ds((tid * 8 + kk) * (WOUT * RW),
                                      WOUT * RW)])

    for d_ in zdescs:
        d_.wait()

    plsc.subcore_barrier()

    flag_vec = flagbuf[...]

    @pl.when(flag_vec[0] > 0)
    def _slow():
        for blk in range(ROWS // 16):
            gvv = gflag[pl.ds(blk * 16, 16)]
            for t in range(16):
                @pl.when(gvv[t] > 0)
                def _one(g=blk * 16 + t):
                    pltpu.async_copy(
                        ones, out.at[idxb.at[g]], sem_s).wait()

    rbase = k * WOUT
    woff = rbase * RW
    for o in range(7):
        other = c * 16 + slot * 8 + (k + 1 + o) % 8
        pltpu.sync_copy(
            mscr.at[pl.ds((other * 8 + k) * (WOUT * RW), WOUT * RW)], tmpb)

        @plsc.parallel_loop(0, (WOUT * RW) // 16, unroll=8)
        def accadd(i):
            v = bbgrid[pl.ds(woff + i * 16, 16)]
            bbgrid[pl.ds(woff + i * 16, 16)] = v + tmpb[pl.ds(i * 16, 16)]

    @plsc.parallel_loop(0, (WOUT * RW) // 16, unroll=8)
    def thresh(i):
        v = bbgrid[pl.ds(woff + i * 16, 16)]
        bbgrid[pl.ds(woff + i * 16, 16)] = jnp.where(v > zerov, onev, zerov)

    prev = None
    for rl in range(WOUT):
        r = rbase + rl
        z = r // RH
        yy = r % RH
        off = base_cell + z * (H * W) + (yy + Y0) * W + X0
        cur = pltpu.async_copy(bbgrid.at[pl.ds((rbase + rl) * RW, RW)],
                               out.at[pl.ds(off, RW)], sem_z)
        if prev is not None and rl % 20 == 19:
            for d_ in prev:
                d_.wait()
            prev = None
        prev = (prev or []) + [cur]
    for d_ in prev or []:
        d_.wait()


def kernel(pointclouds):
    zsrc = jnp.zeros((ZCH,), jnp.float32)
    flat, _ = _voxelize(pointclouds.reshape(-1), zsrc)
    return flat.reshape(B, D, H, W)


# speedup vs baseline: 8.5177x; 8.5177x over previous
import functools

import jax
import jax.numpy as jnp
from jax import lax
from jax.experimental import pallas as pl
from jax.experimental.pallas import tpu as pltpu
from jax.experimental.pallas import tpu_sc as plsc

X_MIN, X_MAX = -51.2, 51.2
Y_MIN, Y_MAX = -51.2, 51.2
Z_MIN = -2.0
STEP = 0.2
D, H, W = 30, 512, 512
B, N = 4, 131072
G = D * H * W
TOTAL = B * G
NTILE = (B * N) // 32
CH = 4096
NCH = NTILE // CH
ZCH = 8192
ZPER = G // 8
NZ = ZPER // ZCH
ROWS = NTILE // 128

RH = 48
RW = 48
Y0 = 232
X0 = 232
RROWS = D * RH
RGW = RROWS * RW
TRASH = RGW
WOUT = (D * RH) // 8
GTRASH = ROWS + 8

_mesh = plsc.VectorSubcoreMesh(core_axis_name="c", subcore_axis_name="s")


@functools.partial(
    pl.kernel,
    out_type=(jax.ShapeDtypeStruct((TOTAL,), jnp.float32),
              jax.ShapeDtypeStruct((256 * WOUT * RW,), jnp.float32)),
    mesh=_mesh,
    scratch_types=[
        pltpu.VMEM((2 * CH,), jnp.float32),
        pltpu.VMEM((2 * CH,), jnp.float32),
        pltpu.VMEM((2 * CH,), jnp.float32),
        pltpu.VMEM((RGW + 16,), jnp.float32),
        pltpu.VMEM((ROWS, 128), jnp.int32),
        pltpu.VMEM((ZCH,), jnp.float32),
        pltpu.VMEM((ROWS + 16,), jnp.int32),
        pltpu.VMEM((WOUT * RW,), jnp.float32),
        pltpu.VMEM((128,), jnp.float32),
        pltpu.VMEM((16,), jnp.int32),
        pltpu.VMEM((16,), jnp.int32),
        pltpu.SemaphoreType.DMA,
        pltpu.SemaphoreType.DMA,
        pltpu.SemaphoreType.DMA,
    ],
    compiler_params=pltpu.CompilerParams(needs_layout_passes=False),
)
def _voxelize(xs, ys, zs, zsrc, out, mscr, xb, yb, zb, bbgrid, idxb, zbuf,
              gflag, tmpb, ones, flagbuf, fbbuf, sem_p, sem_z, sem_s):
    c = lax.axis_index("c")
    s = lax.axis_index("s")
    b = 2 * c + s // 8
    k = s % 8
    slot = s // 8
    pbase = k * NTILE
    base_cell = b * G

    cp_0 = pltpu.async_copy(zsrc, zbuf, sem_z)
    pdesc = [None] * NCH
    pdesc[0] = [
        pltpu.async_copy(src_.at[pl.ds(b * N + pbase, CH)],
                         dst_.at[pl.ds(0, CH)], sem_p)
        for src_, dst_ in ((xs, xb), (ys, yb), (zs, zb))]

    lanes = lax.iota(jnp.int32, 16)
    zeroi = jnp.full((16,), 0, jnp.int32)
    onei = jnp.full((16,), 1, jnp.int32)
    onev = jnp.ones((16,), jnp.float32)
    zerov = jnp.zeros((16,), jnp.float32)
    for i in range(8):
        ones[pl.ds(i * 16, 16)] = onev
    flagbuf[...] = zeroi
    fbbuf[...] = zeroi
    for i in range((ROWS + 16) // 16):
        gflag[pl.ds(i * 16, 16)] = zeroi

    @plsc.parallel_loop(0, (RGW + 16) // 16, unroll=8)
    def bbzero(i):
        bbgrid[pl.ds(i * 16, 16)] = zerov

    cp_0.wait()
    zstart = base_cell + k * ZPER
    zdescs = [
        pltpu.async_copy(zbuf, out.at[pl.ds(zstart + i * ZCH, ZCH)], sem_z)
        for i in range(NZ)
    ]

    sent = jnp.full((16,), -1, jnp.int32)
    trash = jnp.full((16,), TRASH, jnp.int32)
    gtrash = jnp.full((16,), GTRASH, jnp.int32)

    for cc in range(NCH):
        if cc + 1 < NCH:
            pdesc[cc + 1] = [
                pltpu.async_copy(
                    src_.at[pl.ds(b * N + pbase + (cc + 1) * CH, CH)],
                    dst_.at[pl.ds(((cc + 1) % 2) * CH, CH)], sem_p)
                for src_, dst_ in ((xs, xb), (ys, yb), (zs, zb))]
        for d_ in pdesc[cc]:
            d_.wait()
        boff = (cc % 2) * CH

        @plsc.parallel_loop(0, CH // 16, unroll=4)
        def sweep1(jj):
            x = xb[pl.ds(boff + jj * 16, 16)]
            y = yb[pl.ds(boff + jj * 16, 16)]
            z = zb[pl.ds(boff + jj * 16, 16)]
            bx = jnp.clip((x - X_MIN) / STEP, 0.0,
                          float(W - 1)).astype(jnp.int32)
            by = jnp.clip((Y_MAX - y) / STEP, 0.0,
                          float(H - 1)).astype(jnp.int32)
            bz = jnp.clip((z - Z_MIN) / STEP, 0.0,
                          float(D - 1)).astype(jnp.int32)
            m = (x >= X_MIN) & (x <= X_MAX) & (y >= Y_MIN) & (y <= Y_MAX)
            idx = base_cell + ((bz << 18) + (by << 9) + bx)
            idx = jnp.clip(idx, 0, TOTAL - 1)
            ry = by - Y0
            rx = bx - X0
            inside = (m & (ry >= zeroi) & (ry < jnp.full((16,), RH, jnp.int32))
                      & (rx >= zeroi) & (rx < jnp.full((16,), RW, jnp.int32)))
            rg = jnp.clip((bz * RH + ry) * RW + rx, 0, TRASH)
            plsc.store_scatter(bbgrid, [jnp.where(inside, rg, trash)], onev)
            outlier = m & jnp.logical_not(inside)
            jg = cc * (CH // 16) + jj
            g = jg // 8
            gv = jnp.full((16,), g, jnp.int32)
            plsc.store_scatter(gflag, [jnp.where(outlier, gv, gtrash)], onei)
            idxb[g, pl.ds((jg % 8) * 16, 16)] = jnp.where(m, idx, sent)
            av = jnp.where(m, zeroi, onei)
            plsc.store_scatter(flagbuf, [av], onei)
            plsc.store_scatter(fbbuf, [av], idx)

    fb = plsc.load_gather(fbbuf, [zeroi])

    @plsc.parallel_loop(0, NTILE // 16, unroll=8)
    def sweep2(j):
        r = j // 8
        colw = (j % 8) * 16
        v = idxb[r, pl.ds(colw, 16)]
        idxb[r, pl.ds(colw, 16)] = jnp.where(v < zeroi, fb, v)

    tid = c * 16 + s
    for kk in range(8):
        pltpu.sync_copy(bbgrid.at[pl.ds(kk * WOUT * RW, WOUT * RW)],
                        mscr.at[pl.ds((tid * 8 + kk) * (WOUT * RW),
                                      WOUT * RW)])

    for d_ in zdescs:
        d_.wait()

    plsc.subcore_barrier()

    flag_vec = flagbuf[...]

    @pl.when(flag_vec[0] > 0)
    def _slow():
        for blk in range(ROWS // 16):
            gvv = gflag[pl.ds(blk * 16, 16)]
            for t in range(16):
                @pl.when(gvv[t] > 0)
                def _one(g=blk * 16 + t):
                    pltpu.async_copy(
                        ones, out.at[idxb.at[g]], sem_s).wait()

    rbase = k * WOUT
    woff = rbase * RW
    for o in range(7):
        other = c * 16 + slot * 8 + (k + 1 + o) % 8
        pltpu.sync_copy(
            mscr.at[pl.ds((other * 8 + k) * (WOUT * RW), WOUT * RW)], tmpb)

        @plsc.parallel_loop(0, (WOUT * RW) // 16, unroll=8)
        def accadd(i):
            v = bbgrid[pl.ds(woff + i * 16, 16)]
            bbgrid[pl.ds(woff + i * 16, 16)] = v + tmpb[pl.ds(i * 16, 16)]

    @plsc.parallel_loop(0, (WOUT * RW) // 16, unroll=8)
    def thresh(i):
        v = bbgrid[pl.ds(woff + i * 16, 16)]
        bbgrid[pl.ds(woff + i * 16, 16)] = jnp.where(v > zerov, onev, zerov)

    prev = None
    for rl in range(WOUT):
        r = rbase + rl
        z = r // RH
        yy = r % RH
        off = base_cell + z * (H * W) + (yy + Y0) * W + X0
        cur = pltpu.async_copy(bbgrid.at[pl.ds((rbase + rl) * RW, RW)],
                               out.at[pl.ds(off, RW)], sem_z)
        if prev is not None and rl % 20 == 19:
            for d_ in prev:
                d_.wait()
            prev = None
        prev = (prev or []) + [cur]
    for d_ in prev or []:
        d_.wait()


def kernel(pointclouds):
    xs = pointclouds[..., 0].reshape(-1)
    ys = pointclouds[..., 1].reshape(-1)
    zs = pointclouds[..., 2].reshape(-1)
    zsrc = jnp.zeros((ZCH,), jnp.float32)
    flat, _ = _voxelize(xs, ys, zs, zsrc)
    return flat.reshape(B, D, H, W)


# speedup vs baseline: 8.5457x; 1.0033x over previous
import functools

import jax
import jax.numpy as jnp
from jax import lax
from jax.experimental import pallas as pl
from jax.experimental.pallas import tpu as pltpu
from jax.experimental.pallas import tpu_sc as plsc

X_MIN, X_MAX = -51.2, 51.2
Y_MIN, Y_MAX = -51.2, 51.2
Z_MIN = -2.0
STEP = 0.2
D, H, W = 30, 512, 512
B, N = 4, 131072
G = D * H * W
TOTAL = B * G
NTILE = (B * N) // 32
CH = 4096
NCH = NTILE // CH
ZCH = 8192
ZPER = G // 8
NZ = ZPER // ZCH
ROWS = NTILE // 128

RH = 48
RW = 48
Y0 = 232
X0 = 232
RROWS = D * RH
RGW = RROWS * RW
TRASH = RGW
WOUT = (D * RH) // 8
GTRASH = ROWS + 8

_mesh = plsc.VectorSubcoreMesh(core_axis_name="c", subcore_axis_name="s")


@functools.partial(
    pl.kernel,
    out_type=(jax.ShapeDtypeStruct((TOTAL,), jnp.float32),
              jax.ShapeDtypeStruct((256 * WOUT * RW,), jnp.float32)),
    mesh=_mesh,
    scratch_types=[
        pltpu.VMEM((2 * CH,), jnp.float32),
        pltpu.VMEM((2 * CH,), jnp.float32),
        pltpu.VMEM((2 * CH,), jnp.float32),
        pltpu.VMEM((RGW + 16,), jnp.float32),
        pltpu.VMEM((ROWS, 128), jnp.int32),
        pltpu.VMEM((ZCH,), jnp.float32),
        pltpu.VMEM((ROWS + 16,), jnp.int32),
        pltpu.VMEM((WOUT * RW,), jnp.float32),
        pltpu.VMEM((128,), jnp.float32),
        pltpu.VMEM((16,), jnp.int32),
        pltpu.VMEM((16,), jnp.int32),
        pltpu.SemaphoreType.DMA,
        pltpu.SemaphoreType.DMA,
        pltpu.SemaphoreType.DMA,
    ],
    compiler_params=pltpu.CompilerParams(needs_layout_passes=False),
)
def _voxelize(xs, ys, zs, zsrc, out, mscr, xb, yb, zb, bbgrid, idxb, zbuf,
              gflag, tmpb, ones, flagbuf, fbbuf, sem_p, sem_z, sem_s):
    c = lax.axis_index("c")
    s = lax.axis_index("s")
    b = 2 * c + s // 8
    k = s % 8
    slot = s // 8
    pbase = k * NTILE
    base_cell = b * G

    cp_0 = pltpu.async_copy(zsrc, zbuf, sem_z)
    pdesc = [None] * NCH
    pdesc[0] = [
        pltpu.async_copy(src_.at[b, pl.ds(pbase, CH)],
                         dst_.at[pl.ds(0, CH)], sem_p)
        for src_, dst_ in ((xs, xb), (ys, yb), (zs, zb))]

    lanes = lax.iota(jnp.int32, 16)
    zeroi = jnp.full((16,), 0, jnp.int32)
    onei = jnp.full((16,), 1, jnp.int32)
    onev = jnp.ones((16,), jnp.float32)
    zerov = jnp.zeros((16,), jnp.float32)
    for i in range(8):
        ones[pl.ds(i * 16, 16)] = onev
    flagbuf[...] = zeroi
    fbbuf[...] = zeroi
    for i in range((ROWS + 16) // 16):
        gflag[pl.ds(i * 16, 16)] = zeroi

    @plsc.parallel_loop(0, (RGW + 16) // 16, unroll=8)
    def bbzero(i):
        bbgrid[pl.ds(i * 16, 16)] = zerov

    cp_0.wait()
    zstart = base_cell + k * ZPER
    zdescs = [
        pltpu.async_copy(zbuf, out.at[pl.ds(zstart + i * ZCH, ZCH)], sem_z)
        for i in range(NZ)
    ]

    sent = jnp.full((16,), -1, jnp.int32)
    trash = jnp.full((16,), TRASH, jnp.int32)
    gtrash = jnp.full((16,), GTRASH, jnp.int32)

    for cc in range(NCH):
        if cc + 1 < NCH:
            pdesc[cc + 1] = [
                pltpu.async_copy(
                    src_.at[b, pl.ds(pbase + (cc + 1) * CH, CH)],
                    dst_.at[pl.ds(((cc + 1) % 2) * CH, CH)], sem_p)
                for src_, dst_ in ((xs, xb), (ys, yb), (zs, zb))]
        for d_ in pdesc[cc]:
            d_.wait()
        boff = (cc % 2) * CH

        @plsc.parallel_loop(0, CH // 16, unroll=4)
        def sweep1(jj):
            x = xb[pl.ds(boff + jj * 16, 16)]
            y = yb[pl.ds(boff + jj * 16, 16)]
            z = zb[pl.ds(boff + jj * 16, 16)]
            bx = jnp.clip((x - X_MIN) / STEP, 0.0,
                          float(W - 1)).astype(jnp.int32)
            by = jnp.clip((Y_MAX - y) / STEP, 0.0,
                          float(H - 1)).astype(jnp.int32)
            bz = jnp.clip((z - Z_MIN) / STEP, 0.0,
                          float(D - 1)).astype(jnp.int32)
            m = (x >= X_MIN) & (x <= X_MAX) & (y >= Y_MIN) & (y <= Y_MAX)
            idx = base_cell + ((bz << 18) + (by << 9) + bx)
            idx = jnp.clip(idx, 0, TOTAL - 1)
            ry = by - Y0
            rx = bx - X0
            inside = (m & (ry >= zeroi) & (ry < jnp.full((16,), RH, jnp.int32))
                      & (rx >= zeroi) & (rx < jnp.full((16,), RW, jnp.int32)))
            rg = jnp.clip((bz * RH + ry) * RW + rx, 0, TRASH)
            plsc.store_scatter(bbgrid, [jnp.where(inside, rg, trash)], onev)
            outlier = m & jnp.logical_not(inside)
            jg = cc * (CH // 16) + jj
            g = jg // 8
            gv = jnp.full((16,), g, jnp.int32)
            plsc.store_scatter(gflag, [jnp.where(outlier, gv, gtrash)], onei)
            idxb[g, pl.ds((jg % 8) * 16, 16)] = jnp.where(m, idx, sent)
            av = jnp.where(m, zeroi, onei)
            plsc.store_scatter(flagbuf, [av], onei)
            plsc.store_scatter(fbbuf, [av], idx)

    fb = plsc.load_gather(fbbuf, [zeroi])

    @plsc.parallel_loop(0, NTILE // 16, unroll=8)
    def sweep2(j):
        r = j // 8
        colw = (j % 8) * 16
        v = idxb[r, pl.ds(colw, 16)]
        idxb[r, pl.ds(colw, 16)] = jnp.where(v < zeroi, fb, v)

    tid = c * 16 + s
    for kk in range(8):
        pltpu.sync_copy(bbgrid.at[pl.ds(kk * WOUT * RW, WOUT * RW)],
                        mscr.at[pl.ds((tid * 8 + kk) * (WOUT * RW),
                                      WOUT * RW)])

    for d_ in zdescs:
        d_.wait()

    plsc.subcore_barrier()

    flag_vec = flagbuf[...]

    @pl.when(flag_vec[0] > 0)
    def _slow():
        for blk in range(ROWS // 16):
            gvv = gflag[pl.ds(blk * 16, 16)]
            for t in range(16):
                @pl.when(gvv[t] > 0)
                def _one(g=blk * 16 + t):
                    pltpu.async_copy(
                        ones, out.at[idxb.at[g]], sem_s).wait()

    rbase = k * WOUT
    woff = rbase * RW
    for o in range(7):
        other = c * 16 + slot * 8 + (k + 1 + o) % 8
        pltpu.sync_copy(
            mscr.at[pl.ds((other * 8 + k) * (WOUT * RW), WOUT * RW)], tmpb)

        @plsc.parallel_loop(0, (WOUT * RW) // 16, unroll=8)
        def accadd(i):
            v = bbgrid[pl.ds(woff + i * 16, 16)]
            bbgrid[pl.ds(woff + i * 16, 16)] = v + tmpb[pl.ds(i * 16, 16)]

    @plsc.parallel_loop(0, (WOUT * RW) // 16, unroll=8)
    def thresh(i):
        v = bbgrid[pl.ds(woff + i * 16, 16)]
        bbgrid[pl.ds(woff + i * 16, 16)] = jnp.where(v > zerov, onev, zerov)

    prev = None
    for rl in range(WOUT):
        r = rbase + rl
        z = r // RH
        yy = r % RH
        off = base_cell + z * (H * W) + (yy + Y0) * W + X0
        cur = pltpu.async_copy(bbgrid.at[pl.ds((rbase + rl) * RW, RW)],
                               out.at[pl.ds(off, RW)], sem_z)
        if prev is not None and rl % 20 == 19:
            for d_ in prev:
                d_.wait()
            prev = None
        prev = (prev or []) + [cur]
    for d_ in prev or []:
        d_.wait()


def kernel(pointclouds):
    xs = pointclouds[..., 0]
    ys = pointclouds[..., 1]
    zs = pointclouds[..., 2]
    zsrc = jnp.zeros((ZCH,), jnp.float32)
    flat, _ = _voxelize(xs, ys, zs, zsrc)
    return flat.reshape(B, D, H, W)
